# baseline (speedup 1.0000x reference)
import functools

import numpy as np
import jax
import jax.numpy as jnp
from jax import lax
from jax.experimental import pallas as pl
from jax.experimental.pallas import tpu as pltpu
from jax.experimental.pallas import tpu_sc as plsc

_B, _C, _H, _W = 8, 3, 32, 32
_K = 32
_DEPTH = 5
_S = 2 ** _DEPTH
_P = 900
_NCHUNK = 57
_PP = _NCHUNK * 16
_CHW = _C * _H * _W
_BU = 4
_NT = _NCHUNK * (_B // _BU)

_patch_np = np.zeros((_PP,), np.int32)
_ij = np.arange(_P)
_patch_np[:_P] = (_ij // 30) * 32 + (_ij % 30)

_C0 = (0., 0., 0., 0., 0., 0., 0., 0., 1., 1., 1., 1., 1., 1., 1., 1.)
_CA = (0., 0., 1., 1., 0., 0., 1., 1., -1., -1., 0., 0., -1., -1., 0., 0.)
_CB = (0., 0., 0., 0., 1., 1., 1., 1., -1., -1., -1., -1., 0., 0., 0., 0.)
_CAB = (0., 1., -1., 0., -1., 0., -2., -1., 1., 2., 0., 1., 0., 1., -1., 0.)

_LEVEL_N = [2 ** (_DEPTH - lvl) for lvl in range(_DEPTH + 1)]
_NODE_OFF = [int(v) for v in np.concatenate([[0], np.cumsum(_LEVEL_N)[:-1]])]


def _sc_body(x_hbm, ints_hbm, flts_hbm,
             out_hbm,
             xv, iv, pv, cmv, basev, bcast, coeft,
             wv,
             outv):
    k = lax.axis_index("s") * 2 + lax.axis_index("c")

    pltpu.sync_copy(x_hbm, xv)
    pltpu.sync_copy(ints_hbm.at[pl.ds(k * 192, 192)], iv)
    pltpu.sync_copy(ints_hbm.at[pl.ds(_K * 192, _PP)], pv)
    pltpu.sync_copy(flts_hbm.at[pl.ds(k * 1008, 1008)], wv)
    pltpu.sync_copy(flts_hbm.at[pl.ds(_K * 1008, 64)], cmv)

    for side in range(2):
        r = 3 * side
        for half in range(2):
            off = half * 16
            h = iv[pl.ds((r + 0) * 32 + off, 16)]
            w = iv[pl.ds((r + 1) * 32 + off, 16)]
            c = iv[pl.ds((r + 2) * 32 + off, 16)]
            basev[pl.ds(side * 32 + off, 16)] = c * (_H * _W) + h * _W + w

    for q in range(4):
        chunk = basev[pl.ds(q * 16, 16)]
        for i in range(16):
            sel = jnp.full((16,), i, jnp.int32)
            bcast[pl.ds((q * 16 + i) * 16, 16)] = jnp.take(chunk, sel)

    c0v = cmv[pl.ds(0, 16)]
    cav = cmv[pl.ds(16, 16)]
    cbv = cmv[pl.ds(32, 16)]
    cabv = cmv[pl.ds(48, 16)]
    zero = jnp.zeros((16,), jnp.float32)
    for nid in range(63):
        w = wv[pl.ds(nid * 16, 16)]
        e = jnp.exp(w - jnp.max(w))
        en = e / (zero + jnp.sum(e))
        coeft[pl.ds((nid * 4 + 0) * 16, 16)] = zero + jnp.sum(en * c0v)
        coeft[pl.ds((nid * 4 + 1) * 16, 16)] = zero + jnp.sum(en * cav)
        coeft[pl.ds((nid * 4 + 2) * 16, 16)] = zero + jnp.sum(en * cbv)
        coeft[pl.ds((nid * 4 + 3) * 16, 16)] = zero + jnp.sum(en * cabv)

    zi = jnp.zeros((16,), jnp.int32)

    @plsc.parallel_loop(0, _NT)
    def step(t):
        ch = t // 2
        bh = t - ch * 2
        pvec = pv[pl.ds(ch * 16, 16)]
        boffs = [zi + (bh * _BU + i) * _CHW for i in range(_BU)]

        def leaf(s):
            idx0 = bcast[pl.ds(s * 16, 16)] + pvec
            return [plsc.load_gather(xv, [idx0 + bo]) for bo in boffs]

        def node(lvl, j):
            if lvl == 0:
                a = leaf(j)
                b2 = leaf(_S + j)
            else:
                a = node(lvl - 1, 2 * j)
                b2 = node(lvl - 1, 2 * j + 1)
            nid = _NODE_OFF[lvl] + j
            k0 = coeft[pl.ds((nid * 4 + 0) * 16, 16)]
            ka = coeft[pl.ds((nid * 4 + 1) * 16, 16)]
            kb = coeft[pl.ds((nid * 4 + 2) * 16, 16)]
            kab = coeft[pl.ds((nid * 4 + 3) * 16, 16)]
            return [ai * (ka + kab * bi) + (kb * bi + k0)
                    for ai, bi in zip(a, b2)]

        res = node(_DEPTH, 0)
        for i in range(_BU):
            outv[pl.ds((bh * _BU + i) * _PP + ch * 16, 16)] = res[i]

    pltpu.sync_copy(outv, out_hbm.at[k])


_sc_call = functools.partial(
    pl.kernel,
    out_type=jax.ShapeDtypeStruct((_K, _B * _PP), jnp.float32),
    mesh=plsc.VectorSubcoreMesh(core_axis_name="c", subcore_axis_name="s"),
    compiler_params=pltpu.CompilerParams(needs_layout_passes=False),
    scratch_types=[
        pltpu.VMEM((_B * _CHW,), jnp.float32),
        pltpu.VMEM((6 * _S,), jnp.int32),
        pltpu.VMEM((_PP,), jnp.int32),
        pltpu.VMEM((64,), jnp.float32),
        pltpu.VMEM((2 * _S,), jnp.int32),
        pltpu.VMEM((2 * _S * 16,), jnp.int32),
        pltpu.VMEM((63 * 4 * 16,), jnp.float32),
        pltpu.VMEM((63 * 16,), jnp.float32),
        pltpu.VMEM((_B * _PP,), jnp.float32),
    ],
)(_sc_body)


def kernel(x, left_idx, right_idx, W0, W1, W2, W3, W4, W5):
    x2 = x.reshape(_B * _CHW)
    idx6 = jnp.concatenate(
        [jnp.transpose(left_idx[:, 0, :, :], (0, 2, 1)),
         jnp.transpose(right_idx[:, 0, :, :], (0, 2, 1))], axis=1)
    ints = jnp.concatenate([idx6.reshape(-1), jnp.asarray(_patch_np)])
    wall = jnp.concatenate(
        [jnp.transpose(w, (1, 0, 2)).reshape(_K, -1)
         for w in (W0, W1, W2, W3, W4, W5)], axis=1)
    cmat = jnp.asarray(np.stack([_C0, _CA, _CB, _CAB]).astype(np.float32))
    flts = jnp.concatenate([wall.reshape(-1), cmat.reshape(-1)])
    out = _sc_call(x2, ints, flts)
    out = out.reshape(_K, _B, _PP)[:, :, :_P]
    return jnp.transpose(out, (1, 0, 2))[..., None]


# speedup vs baseline: 1.0516x; 1.0516x over previous
import functools

import numpy as np
import jax
import jax.numpy as jnp
from jax import lax
from jax.experimental import pallas as pl
from jax.experimental.pallas import tpu as pltpu
from jax.experimental.pallas import tpu_sc as plsc

_B, _C, _H, _W = 8, 3, 32, 32
_K = 32
_DEPTH = 5
_S = 2 ** _DEPTH
_P = 900
_NCHUNK = 57
_PP = _NCHUNK * 16
_CHW = _C * _H * _W
_BU = 4
_NT = _NCHUNK * (_B // _BU)

_patch_np = np.zeros((_PP,), np.int32)
_ij = np.arange(_P)
_patch_np[:_P] = (_ij // 30) * 32 + (_ij % 30)

_C0 = (0., 0., 0., 0., 0., 0., 0., 0., 1., 1., 1., 1., 1., 1., 1., 1.)
_CA = (0., 0., 1., 1., 0., 0., 1., 1., -1., -1., 0., 0., -1., -1., 0., 0.)
_CB = (0., 0., 0., 0., 1., 1., 1., 1., -1., -1., -1., -1., 0., 0., 0., 0.)
_CAB = (0., 1., -1., 0., -1., 0., -2., -1., 1., 2., 0., 1., 0., 1., -1., 0.)

_LEVEL_N = [2 ** (_DEPTH - lvl) for lvl in range(_DEPTH + 1)]
_NODE_OFF = [int(v) for v in np.concatenate([[0], np.cumsum(_LEVEL_N)[:-1]])]


def _sc_body(x_hbm, ints_hbm, flts_hbm,
             out_hbm,
             xv, iv, pv, basev, bcast, coeft,
             wv,
             outv):
    k = lax.axis_index("s") * 2 + lax.axis_index("c")

    pltpu.sync_copy(x_hbm, xv)
    pltpu.sync_copy(ints_hbm.at[pl.ds(k * 192, 192)], iv)
    pltpu.sync_copy(ints_hbm.at[pl.ds(_K * 192, _PP)], pv)
    pltpu.sync_copy(flts_hbm.at[pl.ds(k * 1024, 1024)], wv)

    for side in range(2):
        r = 3 * side
        for half in range(2):
            off = half * 16
            h = iv[pl.ds((r + 0) * 32 + off, 16)]
            w = iv[pl.ds((r + 1) * 32 + off, 16)]
            c = iv[pl.ds((r + 2) * 32 + off, 16)]
            basev[pl.ds(side * 32 + off, 16)] = c * (_H * _W) + h * _W + w

    for q in range(4):
        chunk = basev[pl.ds(q * 16, 16)]
        for i in range(16):
            sel = jnp.full((16,), i, jnp.int32)
            bcast[pl.ds((q * 16 + i) * 16, 16)] = jnp.take(chunk, sel)

    for q in range(4):
        wg = [wv[pl.ds(g * 64 + q * 16, 16)] for g in range(16)]
        m = wg[0]
        for g in range(1, 16):
            m = jnp.maximum(m, wg[g])
        e = [jnp.exp(w - m) for w in wg]
        s = e[0]
        for g in range(1, 16):
            s = s + e[g]
        en = [ei / s for ei in e]
        k0 = ((en[8] + en[9]) + (en[10] + en[11])) + \
             ((en[12] + en[13]) + (en[14] + en[15]))
        ka = ((en[2] + en[3]) + (en[6] + en[7])) - \
             ((en[8] + en[9]) + (en[12] + en[13]))
        kb = ((en[4] + en[5]) + (en[6] + en[7])) - \
             ((en[8] + en[9]) + (en[10] + en[11]))
        kab = (((en[1] + en[8]) + (en[11] + en[13]))
               - ((en[2] + en[4]) + (en[7] + en[14]))
               + 2.0 * (en[9] - en[6]))
        for i in range(16):
            nid = q * 16 + i
            if nid >= 63:
                break
            sel = jnp.full((16,), i, jnp.int32)
            coeft[pl.ds((nid * 4 + 0) * 16, 16)] = jnp.take(k0, sel)
            coeft[pl.ds((nid * 4 + 1) * 16, 16)] = jnp.take(ka, sel)
            coeft[pl.ds((nid * 4 + 2) * 16, 16)] = jnp.take(kb, sel)
            coeft[pl.ds((nid * 4 + 3) * 16, 16)] = jnp.take(kab, sel)

    zi = jnp.zeros((16,), jnp.int32)

    @plsc.parallel_loop(0, _NT)
    def step(t):
        ch = t // 2
        bh = t - ch * 2
        pvec = pv[pl.ds(ch * 16, 16)]
        boffs = [zi + (bh * _BU + i) * _CHW for i in range(_BU)]

        def leaf(s):
            idx0 = bcast[pl.ds(s * 16, 16)] + pvec
            return [plsc.load_gather(xv, [idx0 + bo]) for bo in boffs]

        def node(lvl, j):
            if lvl == 0:
                a = leaf(j)
                b2 = leaf(_S + j)
            else:
                a = node(lvl - 1, 2 * j)
                b2 = node(lvl - 1, 2 * j + 1)
            nid = _NODE_OFF[lvl] + j
            k0 = coeft[pl.ds((nid * 4 + 0) * 16, 16)]
            ka = coeft[pl.ds((nid * 4 + 1) * 16, 16)]
            kb = coeft[pl.ds((nid * 4 + 2) * 16, 16)]
            kab = coeft[pl.ds((nid * 4 + 3) * 16, 16)]
            return [ai * (ka + kab * bi) + (kb * bi + k0)
                    for ai, bi in zip(a, b2)]

        res = node(_DEPTH, 0)
        for i in range(_BU):
            outv[pl.ds((bh * _BU + i) * _PP + ch * 16, 16)] = res[i]

    pltpu.sync_copy(outv, out_hbm.at[k])


_sc_call = functools.partial(
    pl.kernel,
    out_type=jax.ShapeDtypeStruct((_K, _B * _PP), jnp.float32),
    mesh=plsc.VectorSubcoreMesh(core_axis_name="c", subcore_axis_name="s"),
    compiler_params=pltpu.CompilerParams(needs_layout_passes=False),
    scratch_types=[
        pltpu.VMEM((_B * _CHW,), jnp.float32),
        pltpu.VMEM((6 * _S,), jnp.int32),
        pltpu.VMEM((_PP,), jnp.int32),
        pltpu.VMEM((2 * _S,), jnp.int32),
        pltpu.VMEM((2 * _S * 16,), jnp.int32),
        pltpu.VMEM((63 * 4 * 16,), jnp.float32),
        pltpu.VMEM((16 * 64,), jnp.float32),
        pltpu.VMEM((_B * _PP,), jnp.float32),
    ],
)(_sc_body)


def kernel(x, left_idx, right_idx, W0, W1, W2, W3, W4, W5):
    x2 = x.reshape(_B * _CHW)
    idx6 = jnp.concatenate(
        [jnp.transpose(left_idx[:, 0, :, :], (0, 2, 1)),
         jnp.transpose(right_idx[:, 0, :, :], (0, 2, 1))], axis=1)
    ints = jnp.concatenate([idx6.reshape(-1), jnp.asarray(_patch_np)])
    wg = jnp.concatenate(
        [jnp.transpose(w, (1, 2, 0)) for w in (W0, W1, W2, W3, W4, W5)]
        + [jnp.zeros((_K, 16, 1), jnp.float32)], axis=2)
    flts = wg.reshape(-1)
    out = _sc_call(x2, ints, flts)
    out = out.reshape(_K, _B, _PP)[:, :, :_P]
    return jnp.transpose(out, (1, 0, 2))[..., None]


# speedup vs baseline: 1.0647x; 1.0125x over previous
import functools

import numpy as np
import jax
import jax.numpy as jnp
from jax import lax
from jax.experimental import pallas as pl
from jax.experimental.pallas import tpu as pltpu
from jax.experimental.pallas import tpu_sc as plsc

_B, _C, _H, _W = 8, 3, 32, 32
_K = 32
_DEPTH = 5
_S = 2 ** _DEPTH
_P = 900
_NCHUNK = 57
_PP = _NCHUNK * 16
_CHW = _C * _H * _W
_BU = 4
_NT = _NCHUNK * (_B // _BU)

_patch_np = np.zeros((_PP,), np.int32)
_ij = np.arange(_P)
_patch_np[:_P] = (_ij // 30) * 32 + (_ij % 30)

_C0 = (0., 0., 0., 0., 0., 0., 0., 0., 1., 1., 1., 1., 1., 1., 1., 1.)
_CA = (0., 0., 1., 1., 0., 0., 1., 1., -1., -1., 0., 0., -1., -1., 0., 0.)
_CB = (0., 0., 0., 0., 1., 1., 1., 1., -1., -1., -1., -1., 0., 0., 0., 0.)
_CAB = (0., 1., -1., 0., -1., 0., -2., -1., 1., 2., 0., 1., 0., 1., -1., 0.)

_LEVEL_N = [2 ** (_DEPTH - lvl) for lvl in range(_DEPTH + 1)]
_NODE_OFF = [int(v) for v in np.concatenate([[0], np.cumsum(_LEVEL_N)[:-1]])]


def _sc_body(x_hbm, ints_hbm, flts_hbm,
             out_hbm,
             xv, iv, pv, basev, bcast, coeft,
             wv,
             outv):
    k = lax.axis_index("s") * 2 + lax.axis_index("c")

    pltpu.sync_copy(x_hbm, xv)
    pltpu.sync_copy(ints_hbm.at[pl.ds(k * 192, 192)], iv)
    pltpu.sync_copy(ints_hbm.at[pl.ds(_K * 192, _PP)], pv)
    pltpu.sync_copy(flts_hbm.at[pl.ds(k * 1024, 1024)], wv)

    for side in range(2):
        r = 3 * side
        for half in range(2):
            off = half * 16
            h = iv[pl.ds((r + 0) * 32 + off, 16)]
            w = iv[pl.ds((r + 1) * 32 + off, 16)]
            c = iv[pl.ds((r + 2) * 32 + off, 16)]
            basev[pl.ds(side * 32 + off, 16)] = c * (_H * _W) + h * _W + w

    for q in range(4):
        chunk = basev[pl.ds(q * 16, 16)]
        for i in range(16):
            sel = jnp.full((16,), i, jnp.int32)
            bcast[pl.ds((q * 16 + i) * 16, 16)] = jnp.take(chunk, sel)

    for q in range(4):
        wg = [wv[pl.ds(g * 64 + q * 16, 16)] for g in range(16)]
        m = wg[0]
        for g in range(1, 16):
            m = jnp.maximum(m, wg[g])
        e = [jnp.exp(w - m) for w in wg]
        s = e[0]
        for g in range(1, 16):
            s = s + e[g]
        en = [ei / s for ei in e]
        k0 = ((en[8] + en[9]) + (en[10] + en[11])) + \
             ((en[12] + en[13]) + (en[14] + en[15]))
        ka = ((en[2] + en[3]) + (en[6] + en[7])) - \
             ((en[8] + en[9]) + (en[12] + en[13]))
        kb = ((en[4] + en[5]) + (en[6] + en[7])) - \
             ((en[8] + en[9]) + (en[10] + en[11]))
        kab = (((en[1] + en[8]) + (en[11] + en[13]))
               - ((en[2] + en[4]) + (en[7] + en[14]))
               + 2.0 * (en[9] - en[6]))
        for i in range(16):
            nid = q * 16 + i
            if nid >= 63:
                break
            sel = jnp.full((16,), i, jnp.int32)
            coeft[pl.ds((nid * 4 + 0) * 16, 16)] = jnp.take(k0, sel)
            coeft[pl.ds((nid * 4 + 1) * 16, 16)] = jnp.take(ka, sel)
            coeft[pl.ds((nid * 4 + 2) * 16, 16)] = jnp.take(kb, sel)
            coeft[pl.ds((nid * 4 + 3) * 16, 16)] = jnp.take(kab, sel)

    zi = jnp.zeros((16,), jnp.int32)

    @plsc.parallel_loop(0, _NT)
    def step(t):
        ch = t // 2
        bh = t - ch * 2
        pvec = pv[pl.ds(ch * 16, 16)]
        boffs = [zi + (bh * _BU + i) * _CHW for i in range(_BU)]

        def leaf(s):
            idx0 = bcast[pl.ds(s * 16, 16)] + pvec
            return [plsc.load_gather(xv, [idx0 + bo]) for bo in boffs]

        def node(lvl, j):
            if lvl == 0:
                a = leaf(j)
                b2 = leaf(_S + j)
            else:
                a = node(lvl - 1, 2 * j)
                b2 = node(lvl - 1, 2 * j + 1)
            nid = _NODE_OFF[lvl] + j
            k0 = coeft[pl.ds((nid * 4 + 0) * 16, 16)]
            ka = coeft[pl.ds((nid * 4 + 1) * 16, 16)]
            kb = coeft[pl.ds((nid * 4 + 2) * 16, 16)]
            kab = coeft[pl.ds((nid * 4 + 3) * 16, 16)]
            return [ai * (ka + kab * bi) + (kb * bi + k0)
                    for ai, bi in zip(a, b2)]

        res = node(_DEPTH, 0)
        for i in range(_BU):
            outv[pl.ds((bh * _BU + i) * 1024 + ch * 16, 16)] = res[i]

    for b in range(_B):
        pltpu.sync_copy(outv.at[pl.ds(b * 1024, 1024)], out_hbm.at[b * _K + k])


_sc_call = functools.partial(
    pl.kernel,
    out_type=jax.ShapeDtypeStruct((_B * _K, 1024), jnp.float32),
    mesh=plsc.VectorSubcoreMesh(core_axis_name="c", subcore_axis_name="s"),
    compiler_params=pltpu.CompilerParams(needs_layout_passes=False),
    scratch_types=[
        pltpu.VMEM((_B * _CHW,), jnp.float32),
        pltpu.VMEM((6 * _S,), jnp.int32),
        pltpu.VMEM((_PP,), jnp.int32),
        pltpu.VMEM((2 * _S,), jnp.int32),
        pltpu.VMEM((2 * _S * 16,), jnp.int32),
        pltpu.VMEM((63 * 4 * 16,), jnp.float32),
        pltpu.VMEM((16 * 64,), jnp.float32),
        pltpu.VMEM((_B * 1024,), jnp.float32),
    ],
)(_sc_body)


def kernel(x, left_idx, right_idx, W0, W1, W2, W3, W4, W5):
    x2 = x.reshape(_B * _CHW)
    idx6 = jnp.concatenate(
        [jnp.transpose(left_idx[:, 0, :, :], (0, 2, 1)),
         jnp.transpose(right_idx[:, 0, :, :], (0, 2, 1))], axis=1)
    ints = jnp.concatenate([idx6.reshape(-1), jnp.asarray(_patch_np)])
    wg = jnp.concatenate(
        [jnp.transpose(w, (1, 2, 0)) for w in (W0, W1, W2, W3, W4, W5)]
        + [jnp.zeros((_K, 16, 1), jnp.float32)], axis=2)
    flts = wg.reshape(-1)
    out = _sc_call(x2, ints, flts)
    return out.reshape(_B, _K, 1024)[:, :, :_P, None]


# speedup vs baseline: 1.0667x; 1.0019x over previous
import functools

import numpy as np
import jax
import jax.numpy as jnp
from jax import lax
from jax.experimental import pallas as pl
from jax.experimental.pallas import tpu as pltpu
from jax.experimental.pallas import tpu_sc as plsc

_B, _C, _H, _W = 8, 3, 32, 32
_K = 32
_DEPTH = 5
_S = 2 ** _DEPTH
_P = 900
_NCHUNK = 57
_PP = _NCHUNK * 16
_CHW = _C * _H * _W
_BU = 4
_NT = _NCHUNK * (_B // _BU)

_patch_np = np.zeros((_PP,), np.int32)
_ij = np.arange(_P)
_patch_np[:_P] = (_ij // 30) * 32 + (_ij % 30)

_C0 = (0., 0., 0., 0., 0., 0., 0., 0., 1., 1., 1., 1., 1., 1., 1., 1.)
_CA = (0., 0., 1., 1., 0., 0., 1., 1., -1., -1., 0., 0., -1., -1., 0., 0.)
_CB = (0., 0., 0., 0., 1., 1., 1., 1., -1., -1., -1., -1., 0., 0., 0., 0.)
_CAB = (0., 1., -1., 0., -1., 0., -2., -1., 1., 2., 0., 1., 0., 1., -1., 0.)

_LEVEL_N = [2 ** (_DEPTH - lvl) for lvl in range(_DEPTH + 1)]
_NODE_OFF = [int(v) for v in np.concatenate([[0], np.cumsum(_LEVEL_N)[:-1]])]


def _sc_body(x_hbm, ints_hbm, flts_hbm,
             out_hbm,
             xv, iv, pv, basev, bcast, coeft,
             wv,
             outv):
    k = lax.axis_index("s") * 2 + lax.axis_index("c")

    pltpu.sync_copy(x_hbm, xv)
    pltpu.sync_copy(ints_hbm.at[pl.ds(k * 192, 192)], iv)
    pltpu.sync_copy(ints_hbm.at[pl.ds(_K * 192, _PP)], pv)
    pltpu.sync_copy(flts_hbm.at[pl.ds(k * 1024, 1024)], wv)

    for side in range(2):
        r = 3 * side
        for half in range(2):
            off = half * 16
            h = iv[pl.ds((r + 0) * 32 + off, 16)]
            w = iv[pl.ds((r + 1) * 32 + off, 16)]
            c = iv[pl.ds((r + 2) * 32 + off, 16)]
            basev[pl.ds(side * 32 + off, 16)] = c * (_H * _W) + h * _W + w

    for q in range(4):
        chunk = basev[pl.ds(q * 16, 16)]
        for i in range(16):
            sel = jnp.full((16,), i, jnp.int32)
            bcast[pl.ds((q * 16 + i) * 16, 16)] = jnp.take(chunk, sel)

    for q in range(4):
        wg = [wv[pl.ds(g * 64 + q * 16, 16)] for g in range(16)]
        m = wg[0]
        for g in range(1, 16):
            m = jnp.maximum(m, wg[g])
        e = [jnp.exp(w - m) for w in wg]
        s = e[0]
        for g in range(1, 16):
            s = s + e[g]
        en = [ei / s for ei in e]
        k0 = ((en[8] + en[9]) + (en[10] + en[11])) + \
             ((en[12] + en[13]) + (en[14] + en[15]))
        ka = ((en[2] + en[3]) + (en[6] + en[7])) - \
             ((en[8] + en[9]) + (en[12] + en[13]))
        kb = ((en[4] + en[5]) + (en[6] + en[7])) - \
             ((en[8] + en[9]) + (en[10] + en[11]))
        kab = (((en[1] + en[8]) + (en[11] + en[13]))
               - ((en[2] + en[4]) + (en[7] + en[14]))
               + 2.0 * (en[9] - en[6]))
        for i in range(16):
            nid = q * 16 + i
            if nid >= 63:
                break
            sel = jnp.full((16,), i, jnp.int32)
            coeft[pl.ds((nid * 4 + 0) * 16, 16)] = jnp.take(k0, sel)
            coeft[pl.ds((nid * 4 + 1) * 16, 16)] = jnp.take(ka, sel)
            coeft[pl.ds((nid * 4 + 2) * 16, 16)] = jnp.take(kb, sel)
            coeft[pl.ds((nid * 4 + 3) * 16, 16)] = jnp.take(kab, sel)

    @plsc.parallel_loop(0, _NT)
    def step(t):
        ch = t // 2
        bh = t - ch * 2
        pvec = pv[pl.ds(ch * 16, 16)]
        xbs = [xv.at[pl.ds((bh * _BU + i) * _CHW, _CHW)] for i in range(_BU)]

        def leaf(s):
            idx0 = bcast[pl.ds(s * 16, 16)] + pvec
            return [plsc.load_gather(xb, [idx0]) for xb in xbs]

        def node(lvl, j):
            if lvl == 0:
                a = leaf(j)
                b2 = leaf(_S + j)
            else:
                a = node(lvl - 1, 2 * j)
                b2 = node(lvl - 1, 2 * j + 1)
            nid = _NODE_OFF[lvl] + j
            k0 = coeft[pl.ds((nid * 4 + 0) * 16, 16)]
            ka = coeft[pl.ds((nid * 4 + 1) * 16, 16)]
            kb = coeft[pl.ds((nid * 4 + 2) * 16, 16)]
            kab = coeft[pl.ds((nid * 4 + 3) * 16, 16)]
            return [ai * (ka + kab * bi) + (kb * bi + k0)
                    for ai, bi in zip(a, b2)]

        res = node(_DEPTH, 0)
        for i in range(_BU):
            outv[pl.ds((bh * _BU + i) * 1024 + ch * 16, 16)] = res[i]

    for b in range(_B):
        pltpu.sync_copy(outv.at[pl.ds(b * 1024, 1024)], out_hbm.at[b * _K + k])


_sc_call = functools.partial(
    pl.kernel,
    out_type=jax.ShapeDtypeStruct((_B * _K, 1024), jnp.float32),
    mesh=plsc.VectorSubcoreMesh(core_axis_name="c", subcore_axis_name="s"),
    compiler_params=pltpu.CompilerParams(needs_layout_passes=False),
    scratch_types=[
        pltpu.VMEM((_B * _CHW,), jnp.float32),
        pltpu.VMEM((6 * _S,), jnp.int32),
        pltpu.VMEM((_PP,), jnp.int32),
        pltpu.VMEM((2 * _S,), jnp.int32),
        pltpu.VMEM((2 * _S * 16,), jnp.int32),
        pltpu.VMEM((63 * 4 * 16,), jnp.float32),
        pltpu.VMEM((16 * 64,), jnp.float32),
        pltpu.VMEM((_B * 1024,), jnp.float32),
    ],
)(_sc_body)


def kernel(x, left_idx, right_idx, W0, W1, W2, W3, W4, W5):
    x2 = x.reshape(_B * _CHW)
    idx6 = jnp.concatenate(
        [jnp.transpose(left_idx[:, 0, :, :], (0, 2, 1)),
         jnp.transpose(right_idx[:, 0, :, :], (0, 2, 1))], axis=1)
    ints = jnp.concatenate([idx6.reshape(-1), jnp.asarray(_patch_np)])
    wg = jnp.concatenate(
        [jnp.transpose(w, (1, 2, 0)) for w in (W0, W1, W2, W3, W4, W5)]
        + [jnp.zeros((_K, 16, 1), jnp.float32)], axis=2)
    flts = wg.reshape(-1)
    out = _sc_call(x2, ints, flts)
    return out.reshape(_B, _K, 1024)[:, :, :_P, None]


# speedup vs baseline: 1.0748x; 1.0076x over previous
import functools

import numpy as np
import jax
import jax.numpy as jnp
from jax import lax
from jax.experimental import pallas as pl
from jax.experimental.pallas import tpu as pltpu
from jax.experimental.pallas import tpu_sc as plsc

_B, _C, _H, _W = 8, 3, 32, 32
_K = 32
_DEPTH = 5
_S = 2 ** _DEPTH
_P = 900
_NCHUNK = 57
_PP = _NCHUNK * 16
_CHW = _C * _H * _W
_BU = 4
_NT = _NCHUNK * (_B // _BU)

_patch_np = np.zeros((_PP,), np.int32)
_ij = np.arange(_P)
_patch_np[:_P] = (_ij // 30) * 32 + (_ij % 30)

_C0 = (0., 0., 0., 0., 0., 0., 0., 0., 1., 1., 1., 1., 1., 1., 1., 1.)
_CA = (0., 0., 1., 1., 0., 0., 1., 1., -1., -1., 0., 0., -1., -1., 0., 0.)
_CB = (0., 0., 0., 0., 1., 1., 1., 1., -1., -1., -1., -1., 0., 0., 0., 0.)
_CAB = (0., 1., -1., 0., -1., 0., -2., -1., 1., 2., 0., 1., 0., 1., -1., 0.)

_LEVEL_N = [2 ** (_DEPTH - lvl) for lvl in range(_DEPTH + 1)]
_NODE_OFF = [int(v) for v in np.concatenate([[0], np.cumsum(_LEVEL_N)[:-1]])]


def _sc_body(x_hbm, ints_hbm, flts_hbm,
             out_hbm,
             xv, iv, pv, basev, bcast, coeft,
             wv,
             outv):
    k = lax.axis_index("s") * 2 + lax.axis_index("c")

    pltpu.sync_copy(x_hbm, xv)
    pltpu.sync_copy(ints_hbm.at[pl.ds(k * 192, 192)], iv)
    pltpu.sync_copy(ints_hbm.at[pl.ds(_K * 192, _PP)], pv)
    pltpu.sync_copy(flts_hbm.at[pl.ds(k * 1024, 1024)], wv)

    for side in range(2):
        r = 3 * side
        for half in range(2):
            off = half * 16
            h = iv[pl.ds((r + 0) * 32 + off, 16)]
            w = iv[pl.ds((r + 1) * 32 + off, 16)]
            c = iv[pl.ds((r + 2) * 32 + off, 16)]
            basev[pl.ds(side * 32 + off, 16)] = c * (_H * _W) + h * _W + w

    for q in range(4):
        chunk = basev[pl.ds(q * 16, 16)]
        for i in range(16):
            sel = jnp.full((16,), i, jnp.int32)
            bcast[pl.ds((q * 16 + i) * 16, 16)] = jnp.take(chunk, sel)

    for q in range(4):
        wg = [wv[pl.ds(g * 64 + q * 16, 16)] for g in range(16)]
        m = wg[0]
        for g in range(1, 16):
            m = jnp.maximum(m, wg[g])
        e = [jnp.exp(w - m) for w in wg]
        s = e[0]
        for g in range(1, 16):
            s = s + e[g]
        en = [ei / s for ei in e]
        k0 = ((en[8] + en[9]) + (en[10] + en[11])) + \
             ((en[12] + en[13]) + (en[14] + en[15]))
        ka = ((en[2] + en[3]) + (en[6] + en[7])) - \
             ((en[8] + en[9]) + (en[12] + en[13]))
        kb = ((en[4] + en[5]) + (en[6] + en[7])) - \
             ((en[8] + en[9]) + (en[10] + en[11]))
        kab = (((en[1] + en[8]) + (en[11] + en[13]))
               - ((en[2] + en[4]) + (en[7] + en[14]))
               + 2.0 * (en[9] - en[6]))
        for i in range(16):
            nid = q * 16 + i
            if nid >= 63:
                break
            sel = jnp.full((16,), i, jnp.int32)
            coeft[pl.ds((nid * 4 + 0) * 16, 16)] = jnp.take(k0, sel)
            coeft[pl.ds((nid * 4 + 1) * 16, 16)] = jnp.take(ka, sel)
            coeft[pl.ds((nid * 4 + 2) * 16, 16)] = jnp.take(kb, sel)
            coeft[pl.ds((nid * 4 + 3) * 16, 16)] = jnp.take(kab, sel)

    @plsc.parallel_loop(0, _NT, unroll=2)
    def step(t):
        ch = t // 2
        bh = t - ch * 2
        pvec = pv[pl.ds(ch * 16, 16)]
        xbs = [xv.at[pl.ds((bh * _BU + i) * _CHW, _CHW)] for i in range(_BU)]

        def leaf(s):
            idx0 = bcast[pl.ds(s * 16, 16)] + pvec
            return [plsc.load_gather(xb, [idx0]) for xb in xbs]

        def node(lvl, j):
            if lvl == 0:
                a = leaf(j)
                b2 = leaf(_S + j)
            else:
                a = node(lvl - 1, 2 * j)
                b2 = node(lvl - 1, 2 * j + 1)
            nid = _NODE_OFF[lvl] + j
            k0 = coeft[pl.ds((nid * 4 + 0) * 16, 16)]
            ka = coeft[pl.ds((nid * 4 + 1) * 16, 16)]
            kb = coeft[pl.ds((nid * 4 + 2) * 16, 16)]
            kab = coeft[pl.ds((nid * 4 + 3) * 16, 16)]
            return [ai * (ka + kab * bi) + (kb * bi + k0)
                    for ai, bi in zip(a, b2)]

        res = node(_DEPTH, 0)
        for i in range(_BU):
            outv[pl.ds((bh * _BU + i) * 1024 + ch * 16, 16)] = res[i]

    for b in range(_B):
        pltpu.sync_copy(outv.at[pl.ds(b * 1024, 1024)], out_hbm.at[b * _K + k])


_sc_call = functools.partial(
    pl.kernel,
    out_type=jax.ShapeDtypeStruct((_B * _K, 1024), jnp.float32),
    mesh=plsc.VectorSubcoreMesh(core_axis_name="c", subcore_axis_name="s"),
    compiler_params=pltpu.CompilerParams(needs_layout_passes=False),
    scratch_types=[
        pltpu.VMEM((_B * _CHW,), jnp.float32),
        pltpu.VMEM((6 * _S,), jnp.int32),
        pltpu.VMEM((_PP,), jnp.int32),
        pltpu.VMEM((2 * _S,), jnp.int32),
        pltpu.VMEM((2 * _S * 16,), jnp.int32),
        pltpu.VMEM((63 * 4 * 16,), jnp.float32),
        pltpu.VMEM((16 * 64,), jnp.float32),
        pltpu.VMEM((_B * 1024,), jnp.float32),
    ],
)(_sc_body)


def kernel(x, left_idx, right_idx, W0, W1, W2, W3, W4, W5):
    x2 = x.reshape(_B * _CHW)
    idx6 = jnp.concatenate(
        [jnp.transpose(left_idx[:, 0, :, :], (0, 2, 1)),
         jnp.transpose(right_idx[:, 0, :, :], (0, 2, 1))], axis=1)
    ints = jnp.concatenate([idx6.reshape(-1), jnp.asarray(_patch_np)])
    wg = jnp.concatenate(
        [jnp.transpose(w, (1, 2, 0)) for w in (W0, W1, W2, W3, W4, W5)]
        + [jnp.zeros((_K, 16, 1), jnp.float32)], axis=2)
    flts = wg.reshape(-1)
    out = _sc_call(x2, ints, flts)
    return out.reshape(_B, _K, 1024)[:, :, :_P, None]


# speedup vs baseline: 1.0978x; 1.0214x over previous
import functools

import numpy as np
import jax
import jax.numpy as jnp
from jax import lax
from jax.experimental import pallas as pl
from jax.experimental.pallas import tpu as pltpu
from jax.experimental.pallas import tpu_sc as plsc

_B, _C, _H, _W = 8, 3, 32, 32
_K = 32
_DEPTH = 5
_S = 2 ** _DEPTH
_P = 900
_NCHUNK = 57
_PP = _NCHUNK * 16
_CHW = _C * _H * _W
_BU = 4
_NT = _NCHUNK * (_B // _BU)

_patch_np = np.zeros((_PP,), np.int32)
_ij = np.arange(_P)
_patch_np[:_P] = (_ij // 30) * 32 + (_ij % 30)

_C0 = (0., 0., 0., 0., 0., 0., 0., 0., 1., 1., 1., 1., 1., 1., 1., 1.)
_CA = (0., 0., 1., 1., 0., 0., 1., 1., -1., -1., 0., 0., -1., -1., 0., 0.)
_CB = (0., 0., 0., 0., 1., 1., 1., 1., -1., -1., -1., -1., 0., 0., 0., 0.)
_CAB = (0., 1., -1., 0., -1., 0., -2., -1., 1., 2., 0., 1., 0., 1., -1., 0.)

_LEVEL_N = [2 ** (_DEPTH - lvl) for lvl in range(_DEPTH + 1)]
_NODE_OFF = [int(v) for v in np.concatenate([[0], np.cumsum(_LEVEL_N)[:-1]])]


def _sc_body(x_hbm, ints_hbm, flts_hbm,
             out_hbm,
             xv, iv, pv, basev, bcast, coeft,
             wv,
             outv, xsem):
    k = lax.axis_index("s") * 2 + lax.axis_index("c")

    xcopy = pltpu.async_copy(x_hbm, xv, xsem)
    pltpu.sync_copy(ints_hbm.at[pl.ds(k * 192, 192)], iv)
    pltpu.sync_copy(ints_hbm.at[pl.ds(_K * 192, _PP)], pv)
    pltpu.sync_copy(flts_hbm.at[pl.ds(k * 1024, 1024)], wv)

    for side in range(2):
        r = 3 * side
        for half in range(2):
            off = half * 16
            h = iv[pl.ds((r + 0) * 32 + off, 16)]
            w = iv[pl.ds((r + 1) * 32 + off, 16)]
            c = iv[pl.ds((r + 2) * 32 + off, 16)]
            basev[pl.ds(side * 32 + off, 16)] = c * (_H * _W) + h * _W + w

    for q in range(4):
        chunk = basev[pl.ds(q * 16, 16)]
        for i in range(16):
            sel = jnp.full((16,), i, jnp.int32)
            bcast[pl.ds((q * 16 + i) * 16, 16)] = jnp.take(chunk, sel)

    for q in range(4):
        wg = [wv[pl.ds(g * 64 + q * 16, 16)] for g in range(16)]
        m = wg[0]
        for g in range(1, 16):
            m = jnp.maximum(m, wg[g])
        e = [jnp.exp(w - m) for w in wg]
        s = e[0]
        for g in range(1, 16):
            s = s + e[g]
        en = [ei / s for ei in e]
        k0 = ((en[8] + en[9]) + (en[10] + en[11])) + \
             ((en[12] + en[13]) + (en[14] + en[15]))
        ka = ((en[2] + en[3]) + (en[6] + en[7])) - \
             ((en[8] + en[9]) + (en[12] + en[13]))
        kb = ((en[4] + en[5]) + (en[6] + en[7])) - \
             ((en[8] + en[9]) + (en[10] + en[11]))
        kab = (((en[1] + en[8]) + (en[11] + en[13]))
               - ((en[2] + en[4]) + (en[7] + en[14]))
               + 2.0 * (en[9] - en[6]))
        for i in range(16):
            nid = q * 16 + i
            if nid >= 63:
                break
            sel = jnp.full((16,), i, jnp.int32)
            coeft[pl.ds((nid * 4 + 0) * 16, 16)] = jnp.take(k0, sel)
            coeft[pl.ds((nid * 4 + 1) * 16, 16)] = jnp.take(ka, sel)
            coeft[pl.ds((nid * 4 + 2) * 16, 16)] = jnp.take(kb, sel)
            coeft[pl.ds((nid * 4 + 3) * 16, 16)] = jnp.take(kab, sel)

    xcopy.wait()

    @plsc.parallel_loop(0, _NT, unroll=2)
    def step(t):
        ch = t // 2
        bh = t - ch * 2
        pvec = pv[pl.ds(ch * 16, 16)]
        xbs = [xv.at[pl.ds((bh * _BU + i) * _CHW, _CHW)] for i in range(_BU)]

        def leaf(s):
            idx0 = bcast[pl.ds(s * 16, 16)] + pvec
            return [plsc.load_gather(xb, [idx0]) for xb in xbs]

        def node(lvl, j):
            if lvl == 0:
                a = leaf(j)
                b2 = leaf(_S + j)
            else:
                a = node(lvl - 1, 2 * j)
                b2 = node(lvl - 1, 2 * j + 1)
            nid = _NODE_OFF[lvl] + j
            k0 = coeft[pl.ds((nid * 4 + 0) * 16, 16)]
            ka = coeft[pl.ds((nid * 4 + 1) * 16, 16)]
            kb = coeft[pl.ds((nid * 4 + 2) * 16, 16)]
            kab = coeft[pl.ds((nid * 4 + 3) * 16, 16)]
            return [ai * (ka + kab * bi) + (kb * bi + k0)
                    for ai, bi in zip(a, b2)]

        res = node(_DEPTH, 0)
        for i in range(_BU):
            outv[pl.ds((bh * _BU + i) * 1024 + ch * 16, 16)] = res[i]

    for b in range(_B):
        pltpu.sync_copy(outv.at[pl.ds(b * 1024, 1024)], out_hbm.at[b * _K + k])


_sc_call = functools.partial(
    pl.kernel,
    out_type=jax.ShapeDtypeStruct((_B * _K, 1024), jnp.float32),
    mesh=plsc.VectorSubcoreMesh(core_axis_name="c", subcore_axis_name="s"),
    compiler_params=pltpu.CompilerParams(needs_layout_passes=False),
    scratch_types=[
        pltpu.VMEM((_B * _CHW,), jnp.float32),
        pltpu.VMEM((6 * _S,), jnp.int32),
        pltpu.VMEM((_PP,), jnp.int32),
        pltpu.VMEM((2 * _S,), jnp.int32),
        pltpu.VMEM((2 * _S * 16,), jnp.int32),
        pltpu.VMEM((63 * 4 * 16,), jnp.float32),
        pltpu.VMEM((16 * 64,), jnp.float32),
        pltpu.VMEM((_B * 1024,), jnp.float32),
        pltpu.SemaphoreType.DMA,
    ],
)(_sc_body)


def kernel(x, left_idx, right_idx, W0, W1, W2, W3, W4, W5):
    x2 = x.reshape(_B * _CHW)
    idx6 = jnp.concatenate(
        [jnp.transpose(left_idx[:, 0, :, :], (0, 2, 1)),
         jnp.transpose(right_idx[:, 0, :, :], (0, 2, 1))], axis=1)
    ints = jnp.concatenate([idx6.reshape(-1), jnp.asarray(_patch_np)])
    wg = jnp.concatenate(
        [jnp.transpose(w, (1, 2, 0)) for w in (W0, W1, W2, W3, W4, W5)]
        + [jnp.zeros((_K, 16, 1), jnp.float32)], axis=2)
    flts = wg.reshape(-1)
    out = _sc_call(x2, ints, flts)
    return out.reshape(_B, _K, 1024)[:, :, :_P, None]


# speedup vs baseline: 1.0980x; 1.0002x over previous
import functools

import numpy as np
import jax
import jax.numpy as jnp
from jax import lax
from jax.experimental import pallas as pl
from jax.experimental.pallas import tpu as pltpu
from jax.experimental.pallas import tpu_sc as plsc

_B, _C, _H, _W = 8, 3, 32, 32
_K = 32
_DEPTH = 5
_S = 2 ** _DEPTH
_P = 900
_NCHUNK = 57
_PP = _NCHUNK * 16
_CHW = _C * _H * _W
_BU = 4
_NT = _NCHUNK * (_B // _BU)

_patch_np = np.zeros((_PP,), np.int32)
_ij = np.arange(_P)
_patch_np[:_P] = (_ij // 30) * 32 + (_ij % 30)

_C0 = (0., 0., 0., 0., 0., 0., 0., 0., 1., 1., 1., 1., 1., 1., 1., 1.)
_CA = (0., 0., 1., 1., 0., 0., 1., 1., -1., -1., 0., 0., -1., -1., 0., 0.)
_CB = (0., 0., 0., 0., 1., 1., 1., 1., -1., -1., -1., -1., 0., 0., 0., 0.)
_CAB = (0., 1., -1., 0., -1., 0., -2., -1., 1., 2., 0., 1., 0., 1., -1., 0.)

_LEVEL_N = [2 ** (_DEPTH - lvl) for lvl in range(_DEPTH + 1)]
_NODE_OFF = [int(v) for v in np.concatenate([[0], np.cumsum(_LEVEL_N)[:-1]])]


def _sc_body(x_hbm, flts_hbm,
             out_hbm,
             xv, ivf, pvf, pv, basev, bcast, coeft,
             wv,
             outv, xsem):
    k = lax.axis_index("s") * 2 + lax.axis_index("c")

    xcopy = pltpu.async_copy(x_hbm, xv, xsem)
    pltpu.sync_copy(flts_hbm.at[pl.ds(k * 1024, 1024)], wv)
    pltpu.sync_copy(flts_hbm.at[pl.ds(_K * 1024 + k * 192, 192)], ivf)
    pltpu.sync_copy(flts_hbm.at[pl.ds(_K * 1024 + _K * 192, _PP)], pvf)
    for c in range(_NCHUNK):
        pv[pl.ds(c * 16, 16)] = plsc.bitcast(pvf[pl.ds(c * 16, 16)], jnp.int32)

    def iv(start):
        return plsc.bitcast(ivf[pl.ds(start, 16)], jnp.int32)

    for side in range(2):
        r = 3 * side
        for half in range(2):
            off = half * 16
            h = iv((r + 0) * 32 + off)
            w = iv((r + 1) * 32 + off)
            c = iv((r + 2) * 32 + off)
            basev[pl.ds(side * 32 + off, 16)] = c * (_H * _W) + h * _W + w

    for q in range(4):
        chunk = basev[pl.ds(q * 16, 16)]
        for i in range(16):
            sel = jnp.full((16,), i, jnp.int32)
            bcast[pl.ds((q * 16 + i) * 16, 16)] = jnp.take(chunk, sel)

    for q in range(4):
        wg = [wv[pl.ds(g * 64 + q * 16, 16)] for g in range(16)]
        m = wg[0]
        for g in range(1, 16):
            m = jnp.maximum(m, wg[g])
        e = [jnp.exp(w - m) for w in wg]
        s = e[0]
        for g in range(1, 16):
            s = s + e[g]
        en = [ei / s for ei in e]
        k0 = ((en[8] + en[9]) + (en[10] + en[11])) + \
             ((en[12] + en[13]) + (en[14] + en[15]))
        ka = ((en[2] + en[3]) + (en[6] + en[7])) - \
             ((en[8] + en[9]) + (en[12] + en[13]))
        kb = ((en[4] + en[5]) + (en[6] + en[7])) - \
             ((en[8] + en[9]) + (en[10] + en[11]))
        kab = (((en[1] + en[8]) + (en[11] + en[13]))
               - ((en[2] + en[4]) + (en[7] + en[14]))
               + 2.0 * (en[9] - en[6]))
        for i in range(16):
            nid = q * 16 + i
            if nid >= 63:
                break
            sel = jnp.full((16,), i, jnp.int32)
            coeft[pl.ds((nid * 4 + 0) * 16, 16)] = jnp.take(k0, sel)
            coeft[pl.ds((nid * 4 + 1) * 16, 16)] = jnp.take(ka, sel)
            coeft[pl.ds((nid * 4 + 2) * 16, 16)] = jnp.take(kb, sel)
            coeft[pl.ds((nid * 4 + 3) * 16, 16)] = jnp.take(kab, sel)

    xcopy.wait()

    @plsc.parallel_loop(0, _NT, unroll=2)
    def step(t):
        ch = t // 2
        bh = t - ch * 2
        pvec = pv[pl.ds(ch * 16, 16)]
        xbs = [xv.at[pl.ds((bh * _BU + i) * _CHW, _CHW)] for i in range(_BU)]

        def leaf(s):
            idx0 = bcast[pl.ds(s * 16, 16)] + pvec
            return [plsc.load_gather(xb, [idx0]) for xb in xbs]

        def node(lvl, j):
            if lvl == 0:
                a = leaf(j)
                b2 = leaf(_S + j)
            else:
                a = node(lvl - 1, 2 * j)
                b2 = node(lvl - 1, 2 * j + 1)
            nid = _NODE_OFF[lvl] + j
            k0 = coeft[pl.ds((nid * 4 + 0) * 16, 16)]
            ka = coeft[pl.ds((nid * 4 + 1) * 16, 16)]
            kb = coeft[pl.ds((nid * 4 + 2) * 16, 16)]
            kab = coeft[pl.ds((nid * 4 + 3) * 16, 16)]
            return [ai * (ka + kab * bi) + (kb * bi + k0)
                    for ai, bi in zip(a, b2)]

        res = node(_DEPTH, 0)
        for i in range(_BU):
            outv[pl.ds((bh * _BU + i) * 1024 + ch * 16, 16)] = res[i]

    for b in range(_B):
        pltpu.sync_copy(outv.at[pl.ds(b * 1024, 1024)], out_hbm.at[b * _K + k])


_sc_call = functools.partial(
    pl.kernel,
    out_type=jax.ShapeDtypeStruct((_B * _K, 1024), jnp.float32),
    mesh=plsc.VectorSubcoreMesh(core_axis_name="c", subcore_axis_name="s"),
    compiler_params=pltpu.CompilerParams(needs_layout_passes=False),
    scratch_types=[
        pltpu.VMEM((_B * _CHW,), jnp.float32),
        pltpu.VMEM((6 * _S,), jnp.float32),
        pltpu.VMEM((_PP,), jnp.float32),
        pltpu.VMEM((_PP,), jnp.int32),
        pltpu.VMEM((2 * _S,), jnp.int32),
        pltpu.VMEM((2 * _S * 16,), jnp.int32),
        pltpu.VMEM((63 * 4 * 16,), jnp.float32),
        pltpu.VMEM((16 * 64,), jnp.float32),
        pltpu.VMEM((_B * 1024,), jnp.float32),
        pltpu.SemaphoreType.DMA,
    ],
)(_sc_body)


def kernel(x, left_idx, right_idx, W0, W1, W2, W3, W4, W5):
    x2 = x.reshape(_B * _CHW)
    idx6 = jnp.concatenate(
        [jnp.transpose(left_idx[:, 0, :, :], (0, 2, 1)),
         jnp.transpose(right_idx[:, 0, :, :], (0, 2, 1))], axis=1)
    ints = jnp.concatenate([idx6.reshape(-1), jnp.asarray(_patch_np)])
    wg = jnp.concatenate(
        [jnp.transpose(w, (1, 2, 0)) for w in (W0, W1, W2, W3, W4, W5)]
        + [jnp.zeros((_K, 16, 1), jnp.float32)], axis=2)
    flts = jnp.concatenate(
        [wg.reshape(-1), lax.bitcast_convert_type(ints, jnp.float32)])
    out = _sc_call(x2, flts)
    return out.reshape(_B, _K, 1024)[:, :, :_P, None]
